# Initial kernel scaffold; baseline (speedup 1.0000x reference)
import jax
import jax.numpy as jnp
from jax.experimental import pallas as pl


def kernel(x, word_emb, pos_emb, gamma, beta):
    raise NotImplementedError("write your pallas kernel here")


# speedup vs baseline: 2.8333x; 2.8333x over previous
import functools
import jax
import jax.numpy as jnp
from jax import lax
from jax.experimental import pallas as pl
from jax.experimental.pallas import tpu as pltpu
from jax.experimental.pallas import tpu_sc as plsc

DIM = 128
NREG = DIM // 16
NC = 2
NS = 16
NW = NC * NS
LCHUNK = 100


_GATHER_DN = lax.GatherDimensionNumbers(
    offset_dims=(), collapsed_slice_dims=(0,), start_index_map=(0,)
)


def _permute(x, idx):
    return lax.gather(
        x, idx[:, None], _GATHER_DN, (1,),
        mode=lax.GatherScatterMode.PROMISE_IN_BOUNDS,
    )


def _lane_sum(x, rot_idx):
    for idx in rot_idx:
        x = x + _permute(x, idx)
    return x


def _rsqrt(x):
    i = lax.bitcast_convert_type(x, jnp.int32)
    i = jnp.int32(0x5F3759DF) - lax.shift_right_arithmetic(i, 1)
    y = lax.bitcast_convert_type(i, jnp.float32)
    for _ in range(4):
        y = y * (1.5 - 0.5 * x * y * y)
    return y


def _make_kernel(B, L, eps):
    rows_per_w = B // NW
    nchunk = L // LCHUNK
    mesh = plsc.VectorSubcoreMesh(core_axis_name="c", subcore_axis_name="s")

    @functools.partial(
        pl.kernel,
        mesh=mesh,
        out_type=jax.ShapeDtypeStruct((B, L, DIM), jnp.float32),
        scratch_types=[
            pltpu.VMEM((nchunk, LCHUNK), jnp.int32),
            pltpu.VMEM((L, DIM), jnp.float32),
            pltpu.VMEM((L, DIM), jnp.float32),
            pltpu.VMEM((2, DIM), jnp.float32),
            pltpu.SemaphoreType.DMA,
        ],
    )
    def k(x_hbm, word_hbm, pos_hbm, gamma_hbm, beta_hbm, out_hbm,
          idx_v, rows_v, pos_v, gb_v, sem):
        wid = lax.axis_index("s") * NC + lax.axis_index("c")
        pltpu.sync_copy(pos_hbm.at[pl.ds(0, L)], pos_v)
        pltpu.sync_copy(gamma_hbm, gb_v.at[0])
        pltpu.sync_copy(beta_hbm, gb_v.at[1])
        gamma_r = [gb_v[0, pl.ds(kk * 16, 16)] for kk in range(NREG)]
        beta_r = [gb_v[1, pl.ds(kk * 16, 16)] for kk in range(NREG)]
        lanes = lax.iota(jnp.int32, 16)
        rot_idx = [(lanes + s) & 15 for s in (8, 4, 2, 1)]

        def row_body(i, carry):
            b = wid * rows_per_w + i
            pltpu.sync_copy(x_hbm.at[b], idx_v)
            copies = [
                pltpu.async_copy(
                    word_hbm.at[idx_v.at[j]],
                    rows_v.at[pl.ds(j * LCHUNK, LCHUNK)],
                    sem,
                )
                for j in range(nchunk)
            ]
            for c in copies:
                c.wait()

            def tok_body(t, tcarry):
                e = [
                    rows_v[t, pl.ds(kk * 16, 16)] + pos_v[t, pl.ds(kk * 16, 16)]
                    for kk in range(NREG)
                ]
                s1 = e[0]
                for kk in range(1, NREG):
                    s1 = s1 + e[kk]
                mean = _lane_sum(s1, rot_idx) * (1.0 / DIM)
                d = [ek - mean for ek in e]
                s2 = d[0] * d[0]
                for kk in range(1, NREG):
                    s2 = s2 + d[kk] * d[kk]
                var = _lane_sum(s2, rot_idx) * (1.0 / DIM)
                inv = _rsqrt(var + eps)
                for kk in range(NREG):
                    rows_v[t, pl.ds(kk * 16, 16)] = (
                        d[kk] * (gamma_r[kk] * inv) + beta_r[kk]
                    )
                return tcarry

            lax.fori_loop(0, L, tok_body, 0)
            pltpu.sync_copy(rows_v, out_hbm.at[b])
            return carry

        lax.fori_loop(0, rows_per_w, row_body, 0)

    return k


def kernel(x, word_emb, pos_emb, gamma, beta):
    B, L = x.shape
    x3 = x.reshape(B, L // LCHUNK, LCHUNK)
    k = _make_kernel(B, L, 1e-12)
    return k(x3, word_emb, pos_emb, gamma, beta)


# speedup vs baseline: 5.6794x; 2.0045x over previous
import functools
import jax
import jax.numpy as jnp
from jax import lax
from jax.experimental import pallas as pl
from jax.experimental.pallas import tpu as pltpu
from jax.experimental.pallas import tpu_sc as plsc

DIM = 128
NREG = DIM // 16
NC = 2
NS = 16
NW = NC * NS
LCHUNK = 100
UNROLL = 4

_GATHER_DN = lax.GatherDimensionNumbers(
    offset_dims=(), collapsed_slice_dims=(0,), start_index_map=(0,)
)


def _permute(x, idx):
    return lax.gather(
        x, idx[:, None], _GATHER_DN, (1,),
        mode=lax.GatherScatterMode.PROMISE_IN_BOUNDS,
    )


def _lane_sum(x, rot_idx):
    for idx in rot_idx:
        x = x + _permute(x, idx)
    return x


def _rsqrt(x):
    i = lax.bitcast_convert_type(x, jnp.int32)
    i = jnp.int32(0x5F3759DF) - lax.shift_right_arithmetic(i, 1)
    y = lax.bitcast_convert_type(i, jnp.float32)
    for _ in range(4):
        y = y * (1.5 - 0.5 * x * y * y)
    return y


def _make_kernel(B, L, eps):
    rows_per_w = B // NW
    nchunk = L // LCHUNK
    mesh = plsc.VectorSubcoreMesh(core_axis_name="c", subcore_axis_name="s")

    @functools.partial(
        pl.kernel,
        mesh=mesh,
        out_type=jax.ShapeDtypeStruct((B, L, DIM), jnp.float32),
        scratch_types=[
            pltpu.VMEM((rows_per_w, nchunk, LCHUNK), jnp.int32),
            pltpu.VMEM((L, DIM), jnp.float32),
            pltpu.VMEM((L, DIM), jnp.float32),
            pltpu.VMEM((2, DIM), jnp.float32),
            pltpu.SemaphoreType.DMA,
        ],
    )
    def k(x_hbm, word_hbm, pos_hbm, gamma_hbm, beta_hbm, out_hbm,
          idx_v, rows_v, pos_v, gb_v, sem):
        wid = lax.axis_index("s") * NC + lax.axis_index("c")
        pltpu.sync_copy(x_hbm.at[wid], idx_v)
        pltpu.sync_copy(pos_hbm.at[pl.ds(0, L)], pos_v)
        pltpu.sync_copy(gamma_hbm, gb_v.at[0])
        pltpu.sync_copy(beta_hbm, gb_v.at[1])
        gamma_r = [gb_v[0, pl.ds(kk * 16, 16)] for kk in range(NREG)]
        beta_r = [gb_v[1, pl.ds(kk * 16, 16)] for kk in range(NREG)]
        lanes = lax.iota(jnp.int32, 16)
        rot_idx = [(lanes + s) & 15 for s in (8, 4, 2, 1)]

        def ln_token(t):
            e = [
                rows_v[t, pl.ds(kk * 16, 16)] + pos_v[t, pl.ds(kk * 16, 16)]
                for kk in range(NREG)
            ]
            s1 = e[0]
            for kk in range(1, NREG):
                s1 = s1 + e[kk]
            mean = _lane_sum(s1, rot_idx) * (1.0 / DIM)
            d = [ek - mean for ek in e]
            s2 = d[0] * d[0]
            for kk in range(1, NREG):
                s2 = s2 + d[kk] * d[kk]
            var = _lane_sum(s2, rot_idx) * (1.0 / DIM)
            inv = _rsqrt(var + eps)
            for kk in range(NREG):
                rows_v[t, pl.ds(kk * 16, 16)] = (
                    d[kk] * (gamma_r[kk] * inv) + beta_r[kk]
                )

        def row_body(i, carry):
            b = wid * rows_per_w + i
            copies = [
                pltpu.async_copy(
                    word_hbm.at[idx_v.at[i, j]],
                    rows_v.at[pl.ds(j * LCHUNK, LCHUNK)],
                    sem,
                )
                for j in range(nchunk)
            ]
            for c in copies:
                c.wait()

            def tok_body(tt, tcarry):
                for u in range(UNROLL):
                    ln_token(tt * UNROLL + u)
                return tcarry

            lax.fori_loop(0, L // UNROLL, tok_body, 0)
            pltpu.sync_copy(rows_v, out_hbm.at[b])
            return carry

        lax.fori_loop(0, rows_per_w, row_body, 0)

    return k


def kernel(x, word_emb, pos_emb, gamma, beta):
    B, L = x.shape
    x4 = x.reshape(NW, B // NW, L // LCHUNK, LCHUNK)
    k = _make_kernel(B, L, 1e-12)
    return k(x4, word_emb, pos_emb, gamma, beta)
